# Initial kernel scaffold; baseline (speedup 1.0000x reference)
import jax
import jax.numpy as jnp
from jax.experimental import pallas as pl


def kernel(hidden_states, token_type_ids, position_ids, Wv_qkv, Wl_qkv, Wv_dense, Wl_dense):
    raise NotImplementedError("write your pallas kernel here")


# speedup vs baseline: 1.4151x; 1.4151x over previous
import functools

import jax
import jax.numpy as jnp
from jax.experimental import pallas as pl

B, L, D, H = 1, 2048, 1024, 16
DH = D // H
VISION_TOKEN_TYPE = 1

TLA = 256
TQ = 512
TLC = 256

_NEG_INF = jnp.finfo(jnp.float32).min


def _mask_kernel(tt_ref, vm_ref):
    tt = tt_ref[...]
    nxt = jnp.roll(tt, -1, axis=1)
    col = jax.lax.broadcasted_iota(jnp.int32, (1, L), 1)
    vm = (tt == VISION_TOKEN_TYPE) & (nxt == VISION_TOKEN_TYPE) & (col < L - 1)
    vm_ref[...] = vm.astype(jnp.int32)


def _qkv_kernel(vm_ref, pos_ref, h_ref, wv_ref, wl_ref, q_ref, k_ref, v_ref):
    h = h_ref[0]
    qkv_v = jnp.dot(h, wv_ref[...], preferred_element_type=jnp.float32)
    qkv_l = jnp.dot(h, wl_ref[...], preferred_element_type=jnp.float32)
    vm = jnp.transpose(vm_ref[...]) != 0
    mixed = jnp.where(vm, qkv_v, qkv_l)

    q = mixed[:, :D].reshape(TLA, H, DH)
    k = mixed[:, D:2 * D].reshape(TLA, H, DH)
    v = mixed[:, 2 * D:].reshape(TLA, H, DH)

    pos = pos_ref[...].astype(jnp.float32)
    pos = jnp.transpose(pos)
    exps = jax.lax.broadcasted_iota(jnp.int32, (1, DH // 2), 1).astype(jnp.float32) * (2.0 / DH)
    inv_freq = jnp.exp(exps * (-jnp.log(10000.0)))
    freqs = pos * inv_freq
    emb = jnp.concatenate([freqs, freqs], axis=-1)
    cos = jnp.cos(emb)[:, None, :]
    sin = jnp.sin(emb)[:, None, :]

    def rot_half(x):
        return jnp.concatenate([-x[..., DH // 2:], x[..., :DH // 2]], axis=-1)

    q = q * cos + rot_half(q) * sin
    k = k * cos + rot_half(k) * sin

    q_ref[...] = jnp.transpose(q, (1, 0, 2))
    k_ref[...] = jnp.transpose(k, (1, 0, 2))
    v_ref[...] = jnp.transpose(v, (1, 0, 2))


def _attn_kernel(q_ref, k_ref, v_ref, o_ref):
    qi = pl.program_id(1)
    q = q_ref[0]
    k = k_ref[0]
    v = v_ref[0]
    s = jax.lax.dot_general(q, k, (((1,), (1,)), ((), ())),
                            preferred_element_type=jnp.float32)
    s = s * (1.0 / (DH ** 0.5))
    row = jax.lax.broadcasted_iota(jnp.int32, (TQ, L), 0) + qi * TQ
    col = jax.lax.broadcasted_iota(jnp.int32, (TQ, L), 1)
    s = jnp.where(row >= col, s, _NEG_INF)
    m = jnp.max(s, axis=1, keepdims=True)
    p = jnp.exp(s - m)
    denom = jnp.sum(p, axis=1, keepdims=True)
    o = jnp.dot(p, v, preferred_element_type=jnp.float32) / denom
    o_ref[0] = o


def _out_kernel(vm_ref, c_ref, wv_ref, wl_ref, o_ref):
    c = jnp.transpose(c_ref[...], (1, 0, 2)).reshape(TLC, D)
    ov = jnp.dot(c, wv_ref[...], preferred_element_type=jnp.float32)
    ol = jnp.dot(c, wl_ref[...], preferred_element_type=jnp.float32)
    vm = jnp.transpose(vm_ref[...]) != 0
    o_ref[0] = jnp.where(vm, ov, ol)


def kernel(hidden_states, token_type_ids, position_ids, Wv_qkv, Wl_qkv, Wv_dense, Wl_dense):
    tt = token_type_ids.astype(jnp.int32)
    pos = position_ids.astype(jnp.int32)

    vm = pl.pallas_call(
        _mask_kernel,
        in_specs=[pl.BlockSpec((1, L), lambda: (0, 0))],
        out_specs=pl.BlockSpec((1, L), lambda: (0, 0)),
        out_shape=jax.ShapeDtypeStruct((1, L), jnp.int32),
    )(tt)

    q, k, v = pl.pallas_call(
        _qkv_kernel,
        grid=(L // TLA,),
        in_specs=[
            pl.BlockSpec((1, TLA), lambda i: (0, i)),
            pl.BlockSpec((1, TLA), lambda i: (0, i)),
            pl.BlockSpec((1, TLA, D), lambda i: (0, i, 0)),
            pl.BlockSpec((D, 3 * D), lambda i: (0, 0)),
            pl.BlockSpec((D, 3 * D), lambda i: (0, 0)),
        ],
        out_specs=[
            pl.BlockSpec((H, TLA, DH), lambda i: (0, i, 0)),
            pl.BlockSpec((H, TLA, DH), lambda i: (0, i, 0)),
            pl.BlockSpec((H, TLA, DH), lambda i: (0, i, 0)),
        ],
        out_shape=[
            jax.ShapeDtypeStruct((H, L, DH), jnp.float32),
            jax.ShapeDtypeStruct((H, L, DH), jnp.float32),
            jax.ShapeDtypeStruct((H, L, DH), jnp.float32),
        ],
    )(vm, pos, hidden_states, Wv_qkv, Wl_qkv)

    ctx = pl.pallas_call(
        _attn_kernel,
        grid=(H, L // TQ),
        in_specs=[
            pl.BlockSpec((1, TQ, DH), lambda h, i: (h, i, 0)),
            pl.BlockSpec((1, L, DH), lambda h, i: (h, 0, 0)),
            pl.BlockSpec((1, L, DH), lambda h, i: (h, 0, 0)),
        ],
        out_specs=pl.BlockSpec((1, TQ, DH), lambda h, i: (h, i, 0)),
        out_shape=jax.ShapeDtypeStruct((H, L, DH), jnp.float32),
    )(q, k, v)

    out = pl.pallas_call(
        _out_kernel,
        grid=(L // TLC,),
        in_specs=[
            pl.BlockSpec((1, TLC), lambda i: (0, i)),
            pl.BlockSpec((H, TLC, DH), lambda i: (0, i, 0)),
            pl.BlockSpec((D, D), lambda i: (0, 0)),
            pl.BlockSpec((D, D), lambda i: (0, 0)),
        ],
        out_specs=pl.BlockSpec((1, TLC, D), lambda i: (0, i, 0)),
        out_shape=jax.ShapeDtypeStruct((B, L, D), jnp.float32),
    )(vm, ctx, Wv_dense, Wl_dense)

    return out


# speedup vs baseline: 1.8106x; 1.2794x over previous
import functools

import jax
import jax.numpy as jnp
from jax.experimental import pallas as pl

B, L, D, H = 1, 2048, 1024, 16
DH = D // H
VISION_TOKEN_TYPE = 1

TLA = 256
TQ = 512
TLC = 256

_NEG_INF = jnp.finfo(jnp.float32).min


def _mask_kernel(tt_ref, vm_ref):
    tt = tt_ref[...]
    nxt = jnp.roll(tt, -1, axis=1)
    col = jax.lax.broadcasted_iota(jnp.int32, (1, L), 1)
    vm = (tt == VISION_TOKEN_TYPE) & (nxt == VISION_TOKEN_TYPE) & (col < L - 1)
    vm_ref[...] = vm.astype(jnp.int32)


def _qkv_kernel(vm_ref, pos_ref, h_ref, wv_ref, wl_ref, q_ref, k_ref, v_ref):
    h = h_ref[0]
    qkv_v = jnp.dot(h, wv_ref[...], preferred_element_type=jnp.float32)
    qkv_l = jnp.dot(h, wl_ref[...], preferred_element_type=jnp.float32)
    vm = jnp.transpose(vm_ref[...]) != 0
    mixed = jnp.where(vm, qkv_v, qkv_l)

    q = mixed[:, :D].reshape(TLA, H, DH)
    k = mixed[:, D:2 * D].reshape(TLA, H, DH)
    v = mixed[:, 2 * D:].reshape(TLA, H, DH)

    pos = pos_ref[...].astype(jnp.float32)
    pos = jnp.transpose(pos)
    exps = jax.lax.broadcasted_iota(jnp.int32, (1, DH // 2), 1).astype(jnp.float32) * (2.0 / DH)
    inv_freq = jnp.exp(exps * (-jnp.log(10000.0)))
    freqs = pos * inv_freq
    emb = jnp.concatenate([freqs, freqs], axis=-1)
    cos = jnp.cos(emb)[:, None, :]
    sin = jnp.sin(emb)[:, None, :]

    def rot_half(x):
        return jnp.concatenate([-x[..., DH // 2:], x[..., :DH // 2]], axis=-1)

    q = q * cos + rot_half(q) * sin
    k = k * cos + rot_half(k) * sin

    q_ref[...] = jnp.transpose(q, (1, 0, 2))
    k_ref[...] = jnp.transpose(k, (1, 0, 2))
    v_ref[...] = jnp.transpose(v, (1, 0, 2))


def _attn_kernel(q_ref, k_ref, v_ref, o_ref):
    qi = pl.program_id(1)
    q = q_ref[0] * (1.0 / (DH ** 0.5))
    row = jax.lax.broadcasted_iota(jnp.int32, (TQ, TQ), 0)
    col = jax.lax.broadcasted_iota(jnp.int32, (TQ, TQ), 1)

    def body(j, carry):
        acc, m, l = carry
        k = k_ref[0, pl.ds(j * TQ, TQ), :]
        v = v_ref[0, pl.ds(j * TQ, TQ), :]
        s = jax.lax.dot_general(q, k, (((1,), (1,)), ((), ())),
                                preferred_element_type=jnp.float32)
        s = jnp.where((j < qi) | (row >= col), s, _NEG_INF)
        m_new = jnp.maximum(m, jnp.max(s, axis=1, keepdims=True))
        p = jnp.exp(s - m_new)
        alpha = jnp.exp(m - m_new)
        l = l * alpha + jnp.sum(p, axis=1, keepdims=True)
        acc = acc * alpha + jnp.dot(p, v, preferred_element_type=jnp.float32)
        return acc, m_new, l

    acc0 = jnp.zeros((TQ, DH), jnp.float32)
    m0 = jnp.full((TQ, 1), _NEG_INF, jnp.float32)
    l0 = jnp.zeros((TQ, 1), jnp.float32)
    acc, m, l = jax.lax.fori_loop(0, qi + 1, body, (acc0, m0, l0))
    o_ref[0] = acc / l


def _out_kernel(vm_ref, c_ref, wv_ref, wl_ref, o_ref):
    c = jnp.transpose(c_ref[...], (1, 0, 2)).reshape(TLC, D)
    ov = jnp.dot(c, wv_ref[...], preferred_element_type=jnp.float32)
    ol = jnp.dot(c, wl_ref[...], preferred_element_type=jnp.float32)
    vm = jnp.transpose(vm_ref[...]) != 0
    o_ref[0] = jnp.where(vm, ov, ol)


def kernel(hidden_states, token_type_ids, position_ids, Wv_qkv, Wl_qkv, Wv_dense, Wl_dense):
    tt = token_type_ids.astype(jnp.int32)
    pos = position_ids.astype(jnp.int32)

    vm = pl.pallas_call(
        _mask_kernel,
        in_specs=[pl.BlockSpec((1, L), lambda: (0, 0))],
        out_specs=pl.BlockSpec((1, L), lambda: (0, 0)),
        out_shape=jax.ShapeDtypeStruct((1, L), jnp.int32),
    )(tt)

    q, k, v = pl.pallas_call(
        _qkv_kernel,
        grid=(L // TLA,),
        in_specs=[
            pl.BlockSpec((1, TLA), lambda i: (0, i)),
            pl.BlockSpec((1, TLA), lambda i: (0, i)),
            pl.BlockSpec((1, TLA, D), lambda i: (0, i, 0)),
            pl.BlockSpec((D, 3 * D), lambda i: (0, 0)),
            pl.BlockSpec((D, 3 * D), lambda i: (0, 0)),
        ],
        out_specs=[
            pl.BlockSpec((H, TLA, DH), lambda i: (0, i, 0)),
            pl.BlockSpec((H, TLA, DH), lambda i: (0, i, 0)),
            pl.BlockSpec((H, TLA, DH), lambda i: (0, i, 0)),
        ],
        out_shape=[
            jax.ShapeDtypeStruct((H, L, DH), jnp.float32),
            jax.ShapeDtypeStruct((H, L, DH), jnp.float32),
            jax.ShapeDtypeStruct((H, L, DH), jnp.float32),
        ],
    )(vm, pos, hidden_states, Wv_qkv, Wl_qkv)

    ctx = pl.pallas_call(
        _attn_kernel,
        grid=(H, L // TQ),
        in_specs=[
            pl.BlockSpec((1, TQ, DH), lambda h, i: (h, i, 0)),
            pl.BlockSpec((1, L, DH), lambda h, i: (h, 0, 0)),
            pl.BlockSpec((1, L, DH), lambda h, i: (h, 0, 0)),
        ],
        out_specs=pl.BlockSpec((1, TQ, DH), lambda h, i: (h, i, 0)),
        out_shape=jax.ShapeDtypeStruct((H, L, DH), jnp.float32),
    )(q, k, v)

    out = pl.pallas_call(
        _out_kernel,
        grid=(L // TLC,),
        in_specs=[
            pl.BlockSpec((1, TLC), lambda i: (0, i)),
            pl.BlockSpec((H, TLC, DH), lambda i: (0, i, 0)),
            pl.BlockSpec((D, D), lambda i: (0, 0)),
            pl.BlockSpec((D, D), lambda i: (0, 0)),
        ],
        out_specs=pl.BlockSpec((1, TLC, D), lambda i: (0, i, 0)),
        out_shape=jax.ShapeDtypeStruct((B, L, D), jnp.float32),
    )(vm, ctx, Wv_dense, Wl_dense)

    return out
